# baseline (speedup 1.0000x reference)
import functools

import jax
import jax.numpy as jnp
from jax import lax
from jax.experimental import pallas as pl
from jax.experimental.pallas import tpu as pltpu
from jax.experimental.pallas import tpu_sc as plsc

_CHUNK = 256
_RING = 5
_TBN = 24576


def _transpose_pad(wt):
  d, v = wt.shape
  grid = (v + _TBN - 1) // _TBN

  def body(in_ref, out_ref):
    t = in_ref[...]
    out_ref[...] = jnp.pad(t.T, ((0, 0), (0, d)))

  return pl.pallas_call(
      body,
      grid=(grid,),
      in_specs=[pl.BlockSpec((d, _TBN), lambda i: (0, i))],
      out_specs=pl.BlockSpec((_TBN, 2 * d), lambda i: (i, 0)),
      out_shape=jax.ShapeDtypeStruct((v, 2 * d), jnp.float32),
  )(wt)


@functools.partial(jax.jit, static_argnums=(2, 3, 4, 5))
def _gather(idx, table, nw, nc, n_rows, d):
  per_tile = n_rows // nw
  chunks = per_tile // _CHUNK
  n_outer = chunks // _RING
  mesh = plsc.VectorSubcoreMesh(core_axis_name="c", subcore_axis_name="s")

  @functools.partial(
      pl.kernel,
      mesh=mesh,
      out_type=jax.ShapeDtypeStruct((n_rows, 2 * d), jnp.float32),
      scratch_types=(
          [pltpu.VMEM((chunks, _CHUNK), jnp.int32)]
          + [pltpu.VMEM((_CHUNK, d), jnp.float32) for _ in range(_RING)]
          + [pltpu.SemaphoreType.DMA for _ in range(2 * _RING)]
      ),
      compiler_params=pltpu.CompilerParams(use_tc_tiling_on_sc=False),
  )
  def k(idx_hbm, table_hbm, out_hbm, idx_v, *rest):
    bufs = rest[:_RING]
    gsems = rest[_RING:2 * _RING]
    wsems = rest[2 * _RING:]
    wid = lax.axis_index("s") * nc + lax.axis_index("c")
    pltpu.sync_copy(idx_hbm.at[wid], idx_v)
    row0 = wid * per_tile

    def fire_gather(c, b):
      pltpu.async_copy(table_hbm.at[idx_v.at[c]], bufs[b], gsems[b])

    def wait_gather(c, b):
      pltpu.make_async_copy(table_hbm.at[idx_v.at[c]], bufs[b], gsems[b]).wait()

    def fire_wb(c, b):
      pltpu.async_copy(
          bufs[b],
          out_hbm.at[pl.ds(row0 + c * _CHUNK, _CHUNK), pl.ds(0, d)],
          wsems[b])

    def wait_wb(b):
      pltpu.make_async_copy(
          bufs[b], out_hbm.at[pl.ds(row0, _CHUNK), pl.ds(0, d)],
          wsems[b]).wait()

    for b in range(_RING):
      fire_gather(b, b)
    for b in range(_RING):
      wait_gather(b, b)
      fire_wb(b, b)

    @pl.loop(1, n_outer)
    def _(o):
      for b in range(_RING):
        wait_wb(b)
        fire_gather(o * _RING + b, b)
      for b in range(_RING):
        wait_gather(o * _RING + b, b)
        fire_wb(o * _RING + b, b)

    for b in range(_RING):
      wait_wb(b)

  return k(idx, table)


def kernel(token_ids, weight):
  b, s = token_ids.shape
  v, d = weight.shape
  n_rows = b * s
  info = plsc.get_sparse_core_info()
  nw = info.num_cores * info.num_subcores
  per_tile = n_rows // nw
  assert n_rows == nw * per_tile and per_tile % _CHUNK == 0
  assert (per_tile // _CHUNK) % _RING == 0
  idx = (token_ids.astype(jnp.int32) * 2).reshape(nw, per_tile // _CHUNK,
                                                  _CHUNK)
  wpad = _transpose_pad(weight.T).reshape(2 * v, d)
  out_wide = _gather(idx, wpad, nw, info.num_cores, n_rows, d)
  return out_wide[:, :d].reshape(b, s, d)


# speedup vs baseline: 1.0077x; 1.0077x over previous
import functools

import jax
import jax.numpy as jnp
from jax import lax
from jax.experimental import pallas as pl
from jax.experimental.pallas import tpu as pltpu
from jax.experimental.pallas import tpu_sc as plsc

_CHUNK = 256
_RING = 5
_TBN = 32768


def _transpose_pad(wt):
  d, v = wt.shape
  grid = (v + _TBN - 1) // _TBN

  def body(in_ref, out_ref):
    t = in_ref[...]
    out_ref[...] = jnp.pad(t.T, ((0, 0), (0, d)))

  return pl.pallas_call(
      body,
      grid=(grid,),
      in_specs=[pl.BlockSpec((d, _TBN), lambda i: (0, i))],
      out_specs=pl.BlockSpec((_TBN, 2 * d), lambda i: (i, 0)),
      out_shape=jax.ShapeDtypeStruct((v, 2 * d), jnp.float32),
  )(wt)


@functools.partial(jax.jit, static_argnums=(2, 3, 4, 5))
def _gather(idx, table, nw, nc, n_rows, d):
  per_tile = n_rows // nw
  chunks = per_tile // _CHUNK
  n_outer = chunks // _RING
  mesh = plsc.VectorSubcoreMesh(core_axis_name="c", subcore_axis_name="s")

  @functools.partial(
      pl.kernel,
      mesh=mesh,
      out_type=jax.ShapeDtypeStruct((n_rows, 2 * d), jnp.float32),
      scratch_types=(
          [pltpu.VMEM((chunks, _CHUNK), jnp.int32)]
          + [pltpu.VMEM((_CHUNK, d), jnp.float32) for _ in range(_RING)]
          + [pltpu.SemaphoreType.DMA for _ in range(2 * _RING)]
      ),
      compiler_params=pltpu.CompilerParams(use_tc_tiling_on_sc=False),
  )
  def k(idx_hbm, table_hbm, out_hbm, idx_v, *rest):
    bufs = rest[:_RING]
    gsems = rest[_RING:2 * _RING]
    wsems = rest[2 * _RING:]
    wid = lax.axis_index("s") * nc + lax.axis_index("c")
    pltpu.sync_copy(idx_hbm.at[wid], idx_v)
    row0 = wid * per_tile

    def fire_gather(c, b):
      pltpu.async_copy(table_hbm.at[idx_v.at[c]], bufs[b], gsems[b])

    def wait_gather(c, b):
      pltpu.make_async_copy(table_hbm.at[idx_v.at[c]], bufs[b], gsems[b]).wait()

    def fire_wb(c, b):
      pltpu.async_copy(
          bufs[b],
          out_hbm.at[pl.ds(row0 + c * _CHUNK, _CHUNK), pl.ds(0, d)],
          wsems[b])

    def wait_wb(b):
      pltpu.make_async_copy(
          bufs[b], out_hbm.at[pl.ds(row0, _CHUNK), pl.ds(0, d)],
          wsems[b]).wait()

    for b in range(_RING):
      fire_gather(b, b)
    for b in range(_RING):
      wait_gather(b, b)
      fire_wb(b, b)

    @pl.loop(1, n_outer)
    def _(o):
      for b in range(_RING):
        wait_wb(b)
        fire_gather(o * _RING + b, b)
      for b in range(_RING):
        wait_gather(o * _RING + b, b)
        fire_wb(o * _RING + b, b)

    for b in range(_RING):
      wait_wb(b)

  return k(idx, table)


def kernel(token_ids, weight):
  b, s = token_ids.shape
  v, d = weight.shape
  n_rows = b * s
  info = plsc.get_sparse_core_info()
  nw = info.num_cores * info.num_subcores
  per_tile = n_rows // nw
  assert n_rows == nw * per_tile and per_tile % _CHUNK == 0
  assert (per_tile // _CHUNK) % _RING == 0
  idx = (token_ids.astype(jnp.int32) * 2).reshape(nw, per_tile // _CHUNK,
                                                  _CHUNK)
  wpad = _transpose_pad(weight.T).reshape(2 * v, d)
  out_wide = _gather(idx, wpad, nw, info.num_cores, n_rows, d)
  return out_wide[:, :d].reshape(b, s, d)
